# baseline (speedup 1.0000x reference)
import jax
import jax.numpy as jnp
from jax.experimental import pallas as pl

ACTION_N = 1_000_000
BLOCK = 131_072
GRID = (ACTION_N + BLOCK - 1) // BLOCK

_ROTS_A = (13, 15, 26, 6)
_ROTS_B = (17, 29, 16, 24)


def _bernoulli_block(x_ref, out_ref):
    pid = pl.program_id(0)
    base = (pid * BLOCK).astype(jnp.uint32)
    ctr = base + jax.lax.broadcasted_iota(jnp.uint32, (BLOCK,), 0)

    ks0 = jnp.uint32(0)
    ks1 = jnp.uint32(1)
    ks2 = jnp.uint32(0x1BD11BDA) ^ ks0 ^ ks1
    ks = (ks0, ks1, ks2)

    x0 = jnp.full_like(ctr, ks0)
    x1 = ctr + ks1

    def rotl(v, r):
        return (v << jnp.uint32(r)) | (v >> jnp.uint32(32 - r))

    for g in range(5):
        rots = _ROTS_A if g % 2 == 0 else _ROTS_B
        for r in rots:
            x0 = x0 + x1
            x1 = rotl(x1, r)
            x1 = x1 ^ x0
        x0 = x0 + ks[(g + 1) % 3]
        x1 = x1 + ks[(g + 2) % 3] + jnp.uint32(g + 1)

    bits = x0 ^ x1
    fbits = (bits >> jnp.uint32(9)) | jnp.uint32(0x3F800000)
    rand = jax.lax.bitcast_convert_type(fbits, jnp.float32) - jnp.float32(1.0)

    out_ref[...] = jnp.where(rand < x_ref[...], jnp.float32(1.0),
                             jnp.float32(0.0))


def kernel(x):
    return pl.pallas_call(
        _bernoulli_block,
        out_shape=jax.ShapeDtypeStruct((ACTION_N,), jnp.float32),
        grid=(GRID,),
        in_specs=[pl.BlockSpec((BLOCK,), lambda i: (i,))],
        out_specs=pl.BlockSpec((BLOCK,), lambda i: (i,)),
    )(x)


# speedup vs baseline: 7.3310x; 7.3310x over previous
import jax
import jax.numpy as jnp
from jax.experimental import pallas as pl

ACTION_N = 1_000_000
LANES = 128
ROWS = 984
BLOCK = ROWS * LANES
GRID = 8

_ROTS_A = (13, 15, 26, 6)
_ROTS_B = (17, 29, 16, 24)


def _threefry_bernoulli(ctr, xv):
    ks0 = jnp.uint32(0)
    ks1 = jnp.uint32(1)
    ks2 = jnp.uint32(0x1BD11BDA) ^ ks0 ^ ks1
    ks = (ks0, ks1, ks2)

    x0 = jnp.zeros_like(ctr)
    x1 = ctr + ks1

    def rotl(v, r):
        return (v << jnp.uint32(r)) | (v >> jnp.uint32(32 - r))

    for g in range(5):
        rots = _ROTS_A if g % 2 == 0 else _ROTS_B
        for r in rots:
            x0 = x0 + x1
            x1 = rotl(x1, r)
            x1 = x1 ^ x0
        x0 = x0 + ks[(g + 1) % 3]
        x1 = x1 + ks[(g + 2) % 3] + jnp.uint32(g + 1)

    bits = x0 ^ x1
    fbits = (bits >> jnp.uint32(9)) | jnp.uint32(0x3F800000)
    rand = jax.lax.bitcast_convert_type(fbits, jnp.float32) - jnp.float32(1.0)
    return jnp.where(rand < xv, jnp.float32(1.0), jnp.float32(0.0))


def _bernoulli_block(x_ref, out_ref):
    pid = pl.program_id(0)
    base = (pid * BLOCK).astype(jnp.uint32)
    row = jax.lax.broadcasted_iota(jnp.uint32, (ROWS, LANES), 0)
    lane = jax.lax.broadcasted_iota(jnp.uint32, (ROWS, LANES), 1)
    ctr = base + row * jnp.uint32(LANES) + lane

    xv = x_ref[...].reshape(ROWS, LANES)
    out_ref[...] = _threefry_bernoulli(ctr, xv).reshape(BLOCK)


def kernel(x):
    return pl.pallas_call(
        _bernoulli_block,
        out_shape=jax.ShapeDtypeStruct((ACTION_N,), jnp.float32),
        grid=(GRID,),
        in_specs=[pl.BlockSpec((BLOCK,), lambda i: (i,))],
        out_specs=pl.BlockSpec((BLOCK,), lambda i: (i,)),
    )(x)
